# baseline (speedup 1.0000x reference)
import jax
import jax.numpy as jnp
from jax import lax
from jax.experimental import pallas as pl
from jax.experimental.pallas import tpu as pltpu
from jax.experimental.pallas import tpu_sc as plsc

D = 768
L = 16
DJ = D // L
NC = 2
NS = 16
NW = NC * NS
K = 16
TG = 16


def _rsqrt16(x):
    i = plsc.bitcast(x, jnp.int32)
    y = plsc.bitcast(jnp.int32(0x5F3759DF) - (i >> 1), jnp.float32)
    half = x * 0.5
    for _ in range(3):
        y = y * (1.5 - half * y * y)
    return y


def _body(tid_hbm, pid_hbm, tok_hbm, pos_hbm, gam_hbm, bet_hbm, out_hbm,
          tidx, pidx, rows, prows, obuf, sm, sq, gam_v, bet_v,
          gsem0, gsem1, psem0, psem1, osem0, osem1, jsem):
    gsem = (gsem0, gsem1)
    psem = (psem0, psem1)
    osem = (osem0, osem1)

    wid = lax.axis_index("s") * NC + lax.axis_index("c")
    n_tok = tid_hbm.shape[0]
    per_w = n_tok // NW
    n_chunks = per_w // K
    base = wid * per_w

    pltpu.sync_copy(tid_hbm.at[pl.ds(base, per_w)], tidx)
    pltpu.sync_copy(pid_hbm.at[pl.ds(base, per_w)], pidx)
    cg = pltpu.async_copy(gam_hbm, gam_v, jsem)
    cb = pltpu.async_copy(bet_hbm, bet_v, jsem)

    def fire_gathers(c, bb):
        pltpu.async_copy(tok_hbm.at[tidx.at[pl.ds(c * K, K)]],
                         rows.at[bb], gsem[bb])
        pltpu.async_copy(pos_hbm.at[pidx.at[pl.ds(c * K, K)]],
                         prows.at[bb], psem[bb])

    fire_gathers(0, 0)
    fire_gathers(1, 1)
    cg.wait()
    cb.wait()

    lane0 = lax.iota(jnp.int32, L) == 0

    def compute_chunk(b):
        @plsc.parallel_loop(0, K)
        def token_stats(t):
            ss = [jnp.zeros((L,), jnp.float32) for _ in range(4)]
            qq = [jnp.zeros((L,), jnp.float32) for _ in range(4)]
            for j in range(DJ):
                sl = pl.ds(j * L, L)
                v = rows[b, t, sl] + prows[b, t, sl]
                rows[b, t, sl] = v
                ss[j % 4] = ss[j % 4] + v
                qq[j % 4] = qq[j % 4] + v * v
            s = (ss[0] + ss[1]) + (ss[2] + ss[3])
            q = (qq[0] + qq[1]) + (qq[2] + qq[3])
            tvec = jnp.full((L,), t, jnp.int32)
            plsc.store_scatter(sm, [tvec],
                               jnp.full((L,), jnp.sum(s), jnp.float32),
                               mask=lane0)
            plsc.store_scatter(sq, [tvec],
                               jnp.full((L,), jnp.sum(q), jnp.float32),
                               mask=lane0)

        m = sm[...] * (1.0 / D)
        var = sq[...] * (1.0 / D) - m * m
        rstd = _rsqrt16(var + 1e-5)
        cvec = m * rstd

        for bb in (0, 1):
            @pl.when(b == bb)
            def _():
                rows_b = rows.at[bb]
                obuf_b = obuf.at[bb]
                for g0 in range(0, K, TG):
                    a_regs = [jnp.full((L,), rstd[t], jnp.float32)
                              for t in range(g0, g0 + TG)]
                    c_regs = [jnp.full((L,), cvec[t], jnp.float32)
                              for t in range(g0, g0 + TG)]

                    @plsc.parallel_loop(0, DJ, unroll=2)
                    def jbody(j, g0=g0, a_regs=a_regs, c_regs=c_regs):
                        sl = pl.ds(j * L, L)
                        g = gam_v[sl]
                        be = bet_v[sl]
                        for u in range(TG):
                            v = rows_b[g0 + u, sl]
                            obuf_b[g0 + u, sl] = \
                                (v * a_regs[u] - c_regs[u]) * g + be

    def loop_body(c, _):
        b = c & 1
        for bb in (0, 1):
            @pl.when(b == bb)
            def _():
                pltpu.make_async_copy(
                    tok_hbm.at[tidx.at[pl.ds(c * K, K)]], rows.at[bb],
                    gsem[bb]).wait()
                pltpu.make_async_copy(
                    pos_hbm.at[pidx.at[pl.ds(c * K, K)]], prows.at[bb],
                    psem[bb]).wait()

                @pl.when(c >= 2)
                def _():
                    pltpu.make_async_copy(
                        obuf.at[bb], out_hbm.at[pl.ds(base, K)],
                        osem[bb]).wait()

        compute_chunk(b)

        for bb in (0, 1):
            @pl.when(b == bb)
            def _():
                pltpu.async_copy(obuf.at[bb],
                                 out_hbm.at[pl.ds(base + c * K, K)],
                                 osem[bb])

                @pl.when(c + 2 < n_chunks)
                def _():
                    fire_gathers(c + 2, bb)
        return 0

    lax.fori_loop(0, n_chunks, loop_body, 0)

    for bb in (0, 1):
        pltpu.make_async_copy(
            obuf.at[bb], out_hbm.at[pl.ds(base, K)], osem[bb]).wait()


@jax.jit
def _emb(tid, pid, tok_table, pos_table, ln_gamma, ln_beta):
    n_tok = tid.shape[0]
    mesh = plsc.VectorSubcoreMesh(
        core_axis_name="c", subcore_axis_name="s",
        num_cores=NC, num_subcores=NS)
    per_w = n_tok // NW
    return pl.kernel(
        _body,
        out_type=jax.ShapeDtypeStruct((n_tok, D), jnp.float32),
        mesh=mesh,
        compiler_params=pltpu.CompilerParams(needs_layout_passes=False),
        scratch_types=[
            pltpu.VMEM((per_w,), jnp.int32),
            pltpu.VMEM((per_w,), jnp.int32),
            pltpu.VMEM((2, K, D), jnp.float32),
            pltpu.VMEM((2, K, D), jnp.float32),
            pltpu.VMEM((2, K, D), jnp.float32),
            pltpu.VMEM((L,), jnp.float32),
            pltpu.VMEM((L,), jnp.float32),
            pltpu.VMEM((D,), jnp.float32),
            pltpu.VMEM((D,), jnp.float32),
            pltpu.SemaphoreType.DMA,
            pltpu.SemaphoreType.DMA,
            pltpu.SemaphoreType.DMA,
            pltpu.SemaphoreType.DMA,
            pltpu.SemaphoreType.DMA,
            pltpu.SemaphoreType.DMA,
            pltpu.SemaphoreType.DMA,
        ],
    )(tid, pid, tok_table, pos_table, ln_gamma, ln_beta)


def kernel(token_ids, position_ids, tok_table, pos_table, ln_gamma, ln_beta):
    b, s = token_ids.shape
    tid = token_ids.reshape(-1).astype(jnp.int32)
    pid = position_ids.reshape(-1).astype(jnp.int32)
    out = _emb(tid, pid, tok_table, pos_table, ln_gamma, ln_beta)
    return out.reshape(b, s, D)


# speedup vs baseline: 1.2051x; 1.2051x over previous
import jax
import jax.numpy as jnp
from jax import lax
from jax.experimental import pallas as pl
from jax.experimental.pallas import tpu as pltpu
from jax.experimental.pallas import tpu_sc as plsc

D = 768
L = 16
DJ = D // L
NC = 2
NS = 16
NW = NC * NS
K = 16
TG = 16


def _rsqrt16(x):
    i = plsc.bitcast(x, jnp.int32)
    y = plsc.bitcast(jnp.int32(0x5F3759DF) - (i >> 1), jnp.float32)
    half = x * 0.5
    for _ in range(3):
        y = y * (1.5 - half * y * y)
    return y


def _body(tid_hbm, pid_hbm, tok_hbm, pos_hbm, gam_hbm, bet_hbm, out_hbm,
          tidx, pidx, rows, prows, obuf, sm, sq, gam_v, bet_v,
          gsem0, gsem1, psem0, psem1, osem0, osem1, jsem):
    gsem = (gsem0, gsem1)
    psem = (psem0, psem1)
    osem = (osem0, osem1)

    wid = lax.axis_index("s") * NC + lax.axis_index("c")
    n_tok = tid_hbm.shape[0]
    per_w = n_tok // NW
    n_chunks = per_w // K
    base = wid * per_w

    pltpu.sync_copy(tid_hbm.at[pl.ds(base, per_w)], tidx)
    pltpu.sync_copy(pid_hbm.at[pl.ds(base, per_w)], pidx)
    cg = pltpu.async_copy(gam_hbm, gam_v, jsem)
    cb = pltpu.async_copy(bet_hbm, bet_v, jsem)

    def fire_gathers(c, bb):
        pltpu.async_copy(tok_hbm.at[tidx.at[pl.ds(c * K, K)]],
                         rows.at[bb], gsem[bb])
        pltpu.async_copy(pos_hbm.at[pidx.at[pl.ds(c * K, K)]],
                         prows.at[bb], psem[bb])

    fire_gathers(0, 0)
    fire_gathers(1, 1)
    cg.wait()
    cb.wait()

    lane0 = lax.iota(jnp.int32, L) == 0

    def compute_chunk(b):
        @plsc.parallel_loop(0, K)
        def token_stats(t):
            ss = [jnp.zeros((L,), jnp.float32) for _ in range(4)]
            qq = [jnp.zeros((L,), jnp.float32) for _ in range(4)]
            for j in range(DJ):
                sl = pl.ds(j * L, L)
                v = rows[b, t, sl] + prows[b, t, sl]
                rows[b, t, sl] = v
                ss[j % 4] = ss[j % 4] + v
                qq[j % 4] = qq[j % 4] + v * v
            s = (ss[0] + ss[1]) + (ss[2] + ss[3])
            q = (qq[0] + qq[1]) + (qq[2] + qq[3])
            tvec = jnp.full((L,), t, jnp.int32)
            plsc.store_scatter(sm, [tvec],
                               jnp.full((L,), jnp.sum(s), jnp.float32),
                               mask=lane0)
            plsc.store_scatter(sq, [tvec],
                               jnp.full((L,), jnp.sum(q), jnp.float32),
                               mask=lane0)

        m = sm[...] * (1.0 / D)
        var = sq[...] * (1.0 / D) - m * m
        rstd = _rsqrt16(var + 1e-5)
        cvec = m * rstd

        for bb in (0, 1):
            @pl.when(b == bb)
            def _():
                rows_b = rows.at[bb]
                obuf_b = obuf.at[bb]
                for g0 in range(0, K, TG):
                    a_regs = [jnp.full((L,), rstd[t], jnp.float32)
                              for t in range(g0, g0 + TG)]
                    c_regs = [jnp.full((L,), cvec[t], jnp.float32)
                              for t in range(g0, g0 + TG)]

                    @plsc.parallel_loop(0, DJ)
                    def jbody(j, g0=g0, a_regs=a_regs, c_regs=c_regs):
                        sl = pl.ds(j * L, L)
                        g = gam_v[sl]
                        be = bet_v[sl]
                        for u in range(TG):
                            v = rows_b[g0 + u, sl]
                            obuf_b[g0 + u, sl] = \
                                (v * a_regs[u] - c_regs[u]) * g + be

    def loop_body(c, _):
        b = c & 1
        for bb in (0, 1):
            @pl.when(b == bb)
            def _():
                pltpu.make_async_copy(
                    tok_hbm.at[tidx.at[pl.ds(c * K, K)]], rows.at[bb],
                    gsem[bb]).wait()
                pltpu.make_async_copy(
                    pos_hbm.at[pidx.at[pl.ds(c * K, K)]], prows.at[bb],
                    psem[bb]).wait()

                @pl.when(c >= 2)
                def _():
                    pltpu.make_async_copy(
                        obuf.at[bb], out_hbm.at[pl.ds(base, K)],
                        osem[bb]).wait()

        compute_chunk(b)

        for bb in (0, 1):
            @pl.when(b == bb)
            def _():
                pltpu.async_copy(obuf.at[bb],
                                 out_hbm.at[pl.ds(base + c * K, K)],
                                 osem[bb])

                @pl.when(c + 2 < n_chunks)
                def _():
                    fire_gathers(c + 2, bb)
        return 0

    lax.fori_loop(0, n_chunks, loop_body, 0)

    for bb in (0, 1):
        pltpu.make_async_copy(
            obuf.at[bb], out_hbm.at[pl.ds(base, K)], osem[bb]).wait()


@jax.jit
def _emb(tid, pid, tok_table, pos_table, ln_gamma, ln_beta):
    n_tok = tid.shape[0]
    mesh = plsc.VectorSubcoreMesh(
        core_axis_name="c", subcore_axis_name="s",
        num_cores=NC, num_subcores=NS)
    per_w = n_tok // NW
    return pl.kernel(
        _body,
        out_type=jax.ShapeDtypeStruct((n_tok, D), jnp.float32),
        mesh=mesh,
        compiler_params=pltpu.CompilerParams(needs_layout_passes=False),
        scratch_types=[
            pltpu.VMEM((per_w,), jnp.int32),
            pltpu.VMEM((per_w,), jnp.int32),
            pltpu.VMEM((2, K, D), jnp.float32),
            pltpu.VMEM((2, K, D), jnp.float32),
            pltpu.VMEM((2, K, D), jnp.float32),
            pltpu.VMEM((L,), jnp.float32),
            pltpu.VMEM((L,), jnp.float32),
            pltpu.VMEM((D,), jnp.float32),
            pltpu.VMEM((D,), jnp.float32),
            pltpu.SemaphoreType.DMA,
            pltpu.SemaphoreType.DMA,
            pltpu.SemaphoreType.DMA,
            pltpu.SemaphoreType.DMA,
            pltpu.SemaphoreType.DMA,
            pltpu.SemaphoreType.DMA,
            pltpu.SemaphoreType.DMA,
        ],
    )(tid, pid, tok_table, pos_table, ln_gamma, ln_beta)


def kernel(token_ids, position_ids, tok_table, pos_table, ln_gamma, ln_beta):
    b, s = token_ids.shape
    tid = token_ids.reshape(-1).astype(jnp.int32)
    pid = position_ids.reshape(-1).astype(jnp.int32)
    out = _emb(tid, pid, tok_table, pos_table, ln_gamma, ln_beta)
    return out.reshape(b, s, D)
